# baseline (speedup 1.0000x reference)
import jax
import jax.numpy as jnp
from jax.experimental import pallas as pl
from jax.experimental.pallas import tpu as pltpu


def _fused_kernel(x_ref, w_ref, b_ref, adj_a_ref, adj_b_ref, o_ref, y_scr):
    @pl.when(pl.program_id(0) == 0)
    def _():
        y_scr[...] = jax.lax.dot_general(
            x_ref[...], w_ref[...],
            dimension_numbers=(((1,), (1,)), ((), ())),
            preferred_element_type=jnp.float32,
        ).astype(jnp.bfloat16)

    y = y_scr[...]
    o_ref[0] = (
        jnp.dot(adj_a_ref[0].astype(jnp.bfloat16), y,
                preferred_element_type=jnp.float32)
        + b_ref[...]
    )
    o_ref[1] = (
        jnp.dot(adj_b_ref[0].astype(jnp.bfloat16), y,
                preferred_element_type=jnp.float32)
        + b_ref[...]
    )


def kernel(x, adj, W, b):
    n, d_in = x.shape
    d_out = W.shape[0]

    bm = 200
    half = n // 2
    assert half % bm == 0
    adj_r = adj.reshape(2, half, n)

    g = pl.pallas_call(
        _fused_kernel,
        grid=(half // bm,),
        in_specs=[
            pl.BlockSpec((n, d_in), lambda i: (0, 0)),
            pl.BlockSpec((d_out, d_in), lambda i: (0, 0)),
            pl.BlockSpec((1, d_out), lambda i: (0, 0)),
            pl.BlockSpec((1, bm, n), lambda i: (0, i, 0)),
            pl.BlockSpec((1, bm, n), lambda i: (1, i, 0)),
        ],
        out_specs=pl.BlockSpec((2, bm, d_out), lambda i: (0, i, 0)),
        out_shape=jax.ShapeDtypeStruct((2, half, d_out), jnp.float32),
        scratch_shapes=[pltpu.VMEM((n, d_out), jnp.bfloat16)],
        compiler_params=pltpu.CompilerParams(
            dimension_semantics=("arbitrary",),
        ),
    )(x, W, b.reshape(1, d_out), adj_r, adj_r)
    return g.reshape(n, d_out)


# speedup vs baseline: 1.0209x; 1.0209x over previous
import jax
import jax.numpy as jnp
from jax.experimental import pallas as pl
from jax.experimental.pallas import tpu as pltpu


def _fused_kernel(x_ref, w_ref, b_ref, adj_ref, o_ref, y_scr):
    @pl.when(pl.program_id(0) == 0)
    def _():
        y_scr[...] = jax.lax.dot_general(
            x_ref[...], w_ref[...],
            dimension_numbers=(((1,), (1,)), ((), ())),
            preferred_element_type=jnp.float32,
        )

    o_ref[...] = (
        jnp.dot(adj_ref[...], y_scr[...], preferred_element_type=jnp.float32)
        + b_ref[...]
    )


def kernel(x, adj, W, b):
    n, d_in = x.shape
    d_out = W.shape[0]

    bm = 400
    assert n % bm == 0
    g = pl.pallas_call(
        _fused_kernel,
        grid=(n // bm,),
        in_specs=[
            pl.BlockSpec((n, d_in), lambda i: (0, 0)),
            pl.BlockSpec((d_out, d_in), lambda i: (0, 0)),
            pl.BlockSpec((1, d_out), lambda i: (0, 0)),
            pl.BlockSpec((bm, n), lambda i: (i, 0)),
        ],
        out_specs=pl.BlockSpec((bm, d_out), lambda i: (i, 0)),
        out_shape=jax.ShapeDtypeStruct((n, d_out), jnp.float32),
        scratch_shapes=[pltpu.VMEM((n, d_out), jnp.float32)],
        compiler_params=pltpu.CompilerParams(
            dimension_semantics=("arbitrary",),
        ),
    )(x, W, b.reshape(1, d_out), adj)
    return g
